# baseline (speedup 1.0000x reference)
import functools

import jax
import jax.numpy as jnp
from jax import lax
from jax.experimental import pallas as pl
from jax.experimental.pallas import tpu as pltpu
from jax.experimental.pallas import tpu_sc as plsc

N = 10000
E = 20000
D_EDGE = 16
DP = 128

NC = 2
NS = 16
NW = NC * NS
CHUNK = 128
NCHUNK = 5
EB_TILE = NCHUNK * CHUNK
EP = NW * EB_TILE
NP = 10240
ROWS_TILE = NP // NS
HNP = NP // NC
ACC_CHUNKS = HNP // CHUNK + 1
HNPA = ACC_CHUNKS * CHUNK
NCHUNKS_ALL = EP // CHUNK
CPT = NCHUNKS_ALL // NS
CPB = 2
NBLK = CPT // CPB


RING = 6


def _make_sc_gather():
  mesh = plsc.VectorSubcoreMesh(core_axis_name="c", subcore_axis_name="s",
                                num_cores=1)

  @functools.partial(
      pl.kernel,
      mesh=mesh,
      out_type=jax.ShapeDtypeStruct((NS, CPT, CHUNK, DP), jnp.float32),
      scratch_types=[
          pltpu.VMEM((CPT, CHUNK), jnp.int32),
          pltpu.VMEM((RING, CHUNK, DP), jnp.float32),
          pltpu.SemaphoreType.DMA,
          pltpu.SemaphoreType.DMA,
      ],
  )
  def gather_k(table_hbm, idx_hbm, out_hbm, idx_v, rows_v, sem, sem_w):
    sid = lax.axis_index("s")
    pltpu.sync_copy(idx_hbm.at[sid], idx_v)
    cps = {}
    wbs = {}
    for j in range(RING):
      cps[j] = pltpu.async_copy(table_hbm.at[idx_v.at[j]], rows_v.at[j], sem)
    for j in range(CPT):
      b = j % RING
      cps[j].wait()
      wbs[j] = pltpu.async_copy(rows_v.at[b], out_hbm.at[sid, j], sem_w)
      if j + RING < CPT:
        wbs[j].wait()
        cps[j + RING] = pltpu.async_copy(table_hbm.at[idx_v.at[j + RING]],
                                         rows_v.at[b], sem)
    for j in range(CPT - RING, CPT):
      wbs[j].wait()

  return gather_k


def _make_sc_scatter(d):
  mesh = plsc.VectorSubcoreMesh(core_axis_name="c", subcore_axis_name="s")

  @functools.partial(
      pl.kernel,
      mesh=mesh,
      out_type=jax.ShapeDtypeStruct((NC, ACC_CHUNKS, CHUNK, d), jnp.float32),
      scratch_types=[
          pltpu.VMEM((CPT, CHUNK), jnp.int32),
          pltpu.VMEM((2, CPB, CHUNK, d), jnp.float32),
          pltpu.VMEM_SHARED((HNPA, d), jnp.float32),
          pltpu.SemaphoreType.DMA,
          pltpu.SemaphoreType.DMA,
      ],
  )
  def scatter_k(msg_hbm, idx_hbm, zeros_hbm, out_hbm, idx_v, rows_v, acc_sh,
                sem, sem_a):
    cid = lax.axis_index("c")
    sid = lax.axis_index("s")
    pltpu.sync_copy(zeros_hbm, rows_v.at[0, 0])
    for i in range(3):
      g = sid * 3 + i
      @pl.when(g < ACC_CHUNKS)
      def _():
        pltpu.sync_copy(rows_v.at[0, 0], acc_sh.at[pl.ds(g * CHUNK, CHUNK)])
    pltpu.sync_copy(idx_hbm.at[cid, sid], idx_v)
    cp = pltpu.async_copy(msg_hbm.at[sid, 0], rows_v.at[0], sem)
    plsc.subcore_barrier()
    adds = {}
    for b in range(NBLK):
      cur = b % 2
      cp.wait()
      if b >= 1:
        for a in adds[b - 1]:
          a.wait()
      if b + 1 < NBLK:
        cp = pltpu.async_copy(msg_hbm.at[sid, b + 1], rows_v.at[1 - cur], sem)
      adds[b] = [
          pltpu.async_copy(rows_v.at[cur, j], acc_sh.at[idx_v.at[b * CPB + j]],
                           sem_a, add=True) for j in range(CPB)
      ]
    for a in adds[NBLK - 1]:
      a.wait()
    plsc.subcore_barrier()
    for i in range(3):
      g = sid * 3 + i
      @pl.when(g < ACC_CHUNKS)
      def _():
        pltpu.sync_copy(acc_sh.at[pl.ds(g * CHUNK, CHUNK)], rows_v.at[0, 0])
        pltpu.sync_copy(rows_v.at[0, 0], out_hbm.at[cid, g])

  return scatter_k


def _n_groups(dout):
  p = DP // dout
  return -(-(D_EDGE + 1) // p)


def _make_edge_msg(dout, din=DP, eb=512):
  grid = EP // eb
  ng = _n_groups(dout)
  ncol = ng * DP

  def body(ea_ref, xs_ref, w1_ref, b1_ref, w2_ref, msg_ref):
    hb = jnp.maximum(
        jnp.dot(ea_ref[...].astype(jnp.bfloat16), w1_ref[...],
                preferred_element_type=jnp.float32) + b1_ref[...], 0.0)
    y = jnp.dot(xs_ref[:, :din].astype(jnp.bfloat16), w2_ref[...],
                preferred_element_type=jnp.float32)
    acc = hb[:, :DP] * y[:, :DP]
    for g in range(1, ng):
      acc = acc + hb[:, g * DP:(g + 1) * DP] * y[:, g * DP:(g + 1) * DP]
    msg_ref[...] = acc

  def run(ea, xs, w1b, b1b, w2aug):
    return pl.pallas_call(
        body,
        grid=(grid,),
        in_specs=[
            pl.BlockSpec((eb, D_EDGE), lambda i: (i, 0)),
            pl.BlockSpec((eb, DP), lambda i: (i, 0)),
            pl.BlockSpec((D_EDGE, ncol), lambda i: (0, 0)),
            pl.BlockSpec((1, ncol), lambda i: (0, 0)),
            pl.BlockSpec((din, ncol), lambda i: (0, 0)),
        ],
        out_specs=pl.BlockSpec((eb, DP), lambda i: (i, 0)),
        out_shape=jax.ShapeDtypeStruct((EP, DP), jnp.float32),
    )(ea, xs, w1b, b1b, w2aug)

  return run


def _make_node_update(dout, nb=1000):
  grid = N // nb
  inv = 1.0 / dout

  def body(s_ref, x_ref, root_ref, bias_ref, g_ref, b_ref, out_ref):
    t = s_ref[...]
    w = DP
    while w > dout:
      w //= 2
      t = t[:, :w] + t[:, w:2 * w]
    z = (t + jnp.dot(x_ref[...], root_ref[...],
                     preferred_element_type=jnp.float32) + bias_ref[...])
    r = jnp.maximum(z, 0.0)
    mu = jnp.sum(r, axis=-1, keepdims=True) * inv
    sq = jnp.sum(r * r, axis=-1, keepdims=True) * inv
    var = sq - mu * mu
    out = (r - mu) * jax.lax.rsqrt(var + 1e-5) * g_ref[...] + b_ref[...]
    if dout < DP:
      out = jnp.concatenate(
          [out, jnp.zeros((nb, DP - dout), jnp.float32)], axis=1)
    out_ref[...] = out

  def run(s, x, root, biasr, gr, br):
    return pl.pallas_call(
        body,
        grid=(grid,),
        in_specs=[
            pl.BlockSpec((nb, DP), lambda i: (i, 0)),
            pl.BlockSpec((nb, DP), lambda i: (i, 0)),
            pl.BlockSpec((DP, dout), lambda i: (0, 0)),
            pl.BlockSpec((1, dout), lambda i: (0, 0)),
            pl.BlockSpec((1, dout), lambda i: (0, 0)),
            pl.BlockSpec((1, dout), lambda i: (0, 0)),
        ],
        out_specs=pl.BlockSpec((nb, DP), lambda i: (i, 0)),
        out_shape=jax.ShapeDtypeStruct((N, DP), jnp.float32),
    )(s, x, root, biasr, gr, br)

  return run


def _fold(t, dout):
  w = DP
  while w > dout:
    w //= 2
    t = t[:, :w] + t[:, w:2 * w]
  return t


def _ln(z, g, b, dout):
  r = jnp.maximum(z, 0.0)
  inv = 1.0 / dout
  mu = jnp.sum(r, axis=-1, keepdims=True) * inv
  sq = jnp.sum(r * r, axis=-1, keepdims=True) * inv
  return (r - mu) * jax.lax.rsqrt(sq - mu * mu + 1e-5) * g + b


def _make_final(nb=1000):
  grid = N // nb

  def body(s1_ref, s2_ref, s3_ref, x_ref, root1_ref, bias1_ref, g1_ref,
           bb1_ref, root2_ref, bias2_ref, g2_ref, bb2_ref, root3_ref,
           bias3_ref, w1_ref, b1_ref, w2_ref, b2_ref, emb_ref, logp_ref):
    h1 = _ln(
        _fold(s1_ref[...], 32)
        + jnp.dot(x_ref[...], root1_ref[...],
                  preferred_element_type=jnp.float32) + bias1_ref[...],
        g1_ref[...], bb1_ref[...], 32)
    h2 = _ln(
        _fold(s2_ref[...], 64)
        + jnp.dot(h1, root2_ref[...],
                  preferred_element_type=jnp.float32) + bias2_ref[...],
        g2_ref[...], bb2_ref[...], 64)
    emb = (s3_ref[...]
           + jnp.dot(h2, root3_ref[...],
                     preferred_element_type=jnp.float32) + bias3_ref[...])
    emb_ref[...] = emb
    r = jnp.maximum(emb, 0.0)
    y = jnp.dot(r, w1_ref[...], preferred_element_type=jnp.float32) + b1_ref[...]
    y = jnp.dot(y, w2_ref[...], preferred_element_type=jnp.float32) + b2_ref[...]
    m = jnp.max(y, axis=-1, keepdims=True)
    lse = m + jnp.log(jnp.sum(jnp.exp(y - m), axis=-1, keepdims=True))
    logp_ref[...] = y - lse

  def run(s1, s2, s3, x, root1, bias1, g1, bb1, root2, bias2, g2, bb2,
          root3, bias3, w1, b1r, w2, b2r):
    return pl.pallas_call(
        body,
        grid=(grid,),
        in_specs=[
            pl.BlockSpec((nb, DP), lambda i: (i, 0)),
            pl.BlockSpec((nb, DP), lambda i: (i, 0)),
            pl.BlockSpec((nb, DP), lambda i: (i, 0)),
            pl.BlockSpec((nb, DP), lambda i: (i, 0)),
            pl.BlockSpec((DP, 32), lambda i: (0, 0)),
            pl.BlockSpec((1, 32), lambda i: (0, 0)),
            pl.BlockSpec((1, 32), lambda i: (0, 0)),
            pl.BlockSpec((1, 32), lambda i: (0, 0)),
            pl.BlockSpec((32, 64), lambda i: (0, 0)),
            pl.BlockSpec((1, 64), lambda i: (0, 0)),
            pl.BlockSpec((1, 64), lambda i: (0, 0)),
            pl.BlockSpec((1, 64), lambda i: (0, 0)),
            pl.BlockSpec((64, 128), lambda i: (0, 0)),
            pl.BlockSpec((1, 128), lambda i: (0, 0)),
            pl.BlockSpec((128, 64), lambda i: (0, 0)),
            pl.BlockSpec((1, 64), lambda i: (0, 0)),
            pl.BlockSpec((64, 32), lambda i: (0, 0)),
            pl.BlockSpec((1, 32), lambda i: (0, 0)),
        ],
        out_specs=[
            pl.BlockSpec((nb, 128), lambda i: (i, 0)),
            pl.BlockSpec((nb, 32), lambda i: (i, 0)),
        ],
        out_shape=[
            jax.ShapeDtypeStruct((N, 128), jnp.float32),
            jax.ShapeDtypeStruct((N, 32), jnp.float32),
        ],
    )(s1, s2, s3, x, root1, bias1, g1, bb1, root2, bias2, g2, bb2,
      root3, bias3, w1, b1r, w2, b2r)

  return run


def _make_edge_msg_fused(dout, din, hpw, hpslice, hp_bf16, want_hout,
                         eb=512):
  grid = EP // eb
  ng = _n_groups(dout)
  ncol = ng * DP
  inv = 1.0 / din

  def body(ea_ref, ss_ref, hp_ref, rootp_ref, biasp_ref, gp_ref, bp_ref,
           w1_ref, b1_ref, w2_ref, *outs):
    if want_hout:
      msg_ref, hout_ref = outs
    else:
      msg_ref, = outs
    t = ss_ref[...]
    w = DP
    while w > din:
      w //= 2
      t = t[:, :w] + t[:, w:2 * w]
    hp = hp_ref[:, :hpslice]
    if not hp_bf16:
      hp = hp.astype(jnp.bfloat16)
    z = (t + jnp.dot(hp, rootp_ref[...], preferred_element_type=jnp.float32)
         + biasp_ref[...])
    r = jnp.maximum(z, 0.0)
    mu = jnp.sum(r, axis=-1, keepdims=True) * inv
    sq = jnp.sum(r * r, axis=-1, keepdims=True) * inv
    var = sq - mu * mu
    hsrc = ((r - mu) * jax.lax.rsqrt(var + 1e-5) * gp_ref[...]
            + bp_ref[...]).astype(jnp.bfloat16)
    if want_hout:
      hout_ref[...] = hsrc
    hb = jnp.maximum(
        jnp.dot(ea_ref[...].astype(jnp.bfloat16), w1_ref[...],
                preferred_element_type=jnp.float32) + b1_ref[...], 0.0)
    y = jnp.dot(hsrc, w2_ref[...], preferred_element_type=jnp.float32)
    acc = hb[:, :DP] * y[:, :DP]
    for g in range(1, ng):
      acc = acc + hb[:, g * DP:(g + 1) * DP] * y[:, g * DP:(g + 1) * DP]
    msg_ref[...] = acc

  def run(ea, ss, hp, rootp, biasp, gp, bp, w1b, b1b, w2aug):
    out_specs = [pl.BlockSpec((eb, DP), lambda i: (i, 0))]
    out_shape = [jax.ShapeDtypeStruct((EP, DP), jnp.float32)]
    if want_hout:
      out_specs.append(pl.BlockSpec((eb, din), lambda i: (i, 0)))
      out_shape.append(jax.ShapeDtypeStruct((EP, din), jnp.bfloat16))
    res = pl.pallas_call(
        body,
        grid=(grid,),
        in_specs=[
            pl.BlockSpec((eb, D_EDGE), lambda i: (i, 0)),
            pl.BlockSpec((eb, DP), lambda i: (i, 0)),
            pl.BlockSpec((eb, hpw), lambda i: (i, 0)),
            pl.BlockSpec((hpslice, din), lambda i: (0, 0)),
            pl.BlockSpec((1, din), lambda i: (0, 0)),
            pl.BlockSpec((1, din), lambda i: (0, 0)),
            pl.BlockSpec((1, din), lambda i: (0, 0)),
            pl.BlockSpec((D_EDGE, ncol), lambda i: (0, 0)),
            pl.BlockSpec((1, ncol), lambda i: (0, 0)),
            pl.BlockSpec((din, ncol), lambda i: (0, 0)),
        ],
        out_specs=out_specs,
        out_shape=out_shape,
    )(ea, ss, hp, rootp, biasp, gp, bp, w1b, b1b, w2aug)
    return res if want_hout else res[0]

  return run


def _w1aug(w1, b1, dout):
  p = DP // dout
  ng = _n_groups(dout)
  we = jnp.pad(jnp.pad(w1, ((0, 0), (0, 1))), ((0, 0), (0, ng * p - 17)))
  wb = jnp.broadcast_to(we[:, :, None], (D_EDGE, ng * p, dout))
  be = jnp.pad(jnp.concatenate([b1, jnp.ones((1,), jnp.float32)]),
               (0, ng * p - 17))
  bb = jnp.broadcast_to(be[:, None], (ng * p, dout))
  return (wb.reshape(D_EDGE, ng * DP).astype(jnp.bfloat16),
          bb.reshape(1, ng * DP))


def _w2aug(w2, b2, din, dout):
  p = DP // dout
  ng = _n_groups(dout)
  w = w2.reshape(D_EDGE, din, dout).transpose(1, 0, 2)
  w = jnp.concatenate([w, b2.reshape(din, 1, dout)], axis=1)
  w = jnp.pad(w, ((0, 0), (0, ng * p - 17), (0, 0)))
  return w.reshape(din, ng * DP).astype(jnp.bfloat16)


def _pad_cols(v, width=DP):
  return jnp.pad(v, ((0, 0), (0, width - v.shape[1])))


def _pad_vec(v, width=DP):
  return jnp.pad(v.reshape(1, -1), ((0, 0), (0, width - v.shape[0])))


def kernel(x, edge_index, edge_attr,
           en1_W1, en1_b1, en1_W2, en1_b2, root1, bias1, ln1_g, ln1_b,
           en2_W1, en2_b1, en2_W2, en2_b2, root2, bias2, ln2_g, ln2_b,
           en3_W1, en3_b1, en3_W2, en3_b2, root3, bias3,
           pm_W1, pm_b1, pm_W2, pm_b2):
  pad = EP - E
  srcf = jnp.concatenate(
      [edge_index[0].astype(jnp.int32), jnp.zeros((pad,), jnp.int32)])
  src = srcf.reshape(NS, CPT, CHUNK)
  src2 = (srcf + CHUNK * (srcf >= HNP).astype(jnp.int32)).reshape(
      NS, CPT, CHUNK)
  dst = jnp.concatenate(
      [edge_index[1].astype(jnp.int32),
       jnp.full((pad,), N, jnp.int32)])
  trash = HNP + (jnp.arange(EP, dtype=jnp.int32) % CHUNK)
  dstc = jnp.stack([
      jnp.where((dst >= c * HNP) & (dst < (c + 1) * HNP), dst - c * HNP,
                trash) for c in range(NC)
  ]).reshape(NC, NS, CPT, CHUNK)
  ea = jnp.concatenate([edge_attr, jnp.zeros((pad, D_EDGE), jnp.float32)])

  gather = _make_sc_gather()

  def r2(v):
    return v.reshape(1, -1)

  scatter = _make_sc_scatter(DP)
  zeros = jnp.zeros((CHUNK, DP), jnp.float32)

  def agg(msg):
    s = scatter(msg.reshape(NS, NBLK, CPB, CHUNK, DP), dstc, zeros)
    return (s[:, :HNP // CHUNK].reshape(NP, DP)[:N],
            s.reshape(NC * ACC_CHUNKS * CHUNK, DP))

  xp = _pad_cols(x)

  xs1 = gather(xp, src).reshape(EP, DP)
  msg = _make_edge_msg(32, 64)(ea, xs1, *_w1aug(en1_W1, en1_b1, 32),
                               _w2aug(en1_W2, en1_b2, 64, 32))
  sn1, sf1 = agg(msg)
  ss1 = gather(sf1, src2).reshape(EP, DP)

  msg, h1s = _make_edge_msg_fused(64, 32, DP, 64, False, True)(
      ea, ss1, xs1, root1.astype(jnp.bfloat16), r2(bias1), r2(ln1_g),
      r2(ln1_b), *_w1aug(en2_W1, en2_b1, 64),
      _w2aug(en2_W2, en2_b2, 32, 64))
  sn2, sf2 = agg(msg)
  ss2 = gather(sf2, src2).reshape(EP, DP)

  msg = _make_edge_msg_fused(128, 64, 32, 32, True, False)(
      ea, ss2, h1s, root2.astype(jnp.bfloat16), r2(bias2), r2(ln2_g),
      r2(ln2_b), *_w1aug(en3_W1, en3_b1, 128),
      _w2aug(en3_W2, en3_b2, 64, 128))
  sn3, _ = agg(msg)
  root1p = jnp.pad(root1, ((0, DP - 64), (0, 0)))
  emb, logp = _make_final()(
      sn1, sn2, sn3, xp, root1p, r2(bias1), r2(ln1_g), r2(ln1_b),
      root2, r2(bias2), r2(ln2_g), r2(ln2_b), root3, r2(bias3),
      pm_W1, r2(pm_b1), pm_W2, r2(pm_b2))
  return (emb, logp)


# speedup vs baseline: 1.0604x; 1.0604x over previous
import functools

import jax
import jax.numpy as jnp
from jax import lax
from jax.experimental import pallas as pl
from jax.experimental.pallas import tpu as pltpu
from jax.experimental.pallas import tpu_sc as plsc

N = 10000
E = 20000
D_EDGE = 16
DP = 128

NC = 2
NS = 16
NW = NC * NS
CHUNK = 128
NCHUNK = 5
EB_TILE = NCHUNK * CHUNK
EP = NW * EB_TILE
NP = 10240
ROWS_TILE = NP // NS
HNP = NP // NC
ACC_CHUNKS = HNP // CHUNK + 1
HNPA = ACC_CHUNKS * CHUNK
NCHUNKS_ALL = EP // CHUNK
CPT = NCHUNKS_ALL // NS
CPB = 2
NBLK = CPT // CPB


RING = 6


def _make_sc_gather():
  mesh = plsc.VectorSubcoreMesh(core_axis_name="c", subcore_axis_name="s",
                                num_cores=1)

  @functools.partial(
      pl.kernel,
      mesh=mesh,
      out_type=jax.ShapeDtypeStruct((NS, CPT, CHUNK, DP), jnp.float32),
      scratch_types=[
          pltpu.VMEM((CPT, CHUNK), jnp.int32),
          pltpu.VMEM((RING, CHUNK, DP), jnp.float32),
          pltpu.SemaphoreType.DMA,
          pltpu.SemaphoreType.DMA,
      ],
  )
  def gather_k(table_hbm, idx_hbm, out_hbm, idx_v, rows_v, sem, sem_w):
    sid = lax.axis_index("s")
    pltpu.sync_copy(idx_hbm.at[sid], idx_v)
    cps = {}
    wbs = {}
    for j in range(RING):
      cps[j] = pltpu.async_copy(table_hbm.at[idx_v.at[j]], rows_v.at[j], sem)
    for j in range(CPT):
      b = j % RING
      cps[j].wait()
      wbs[j] = pltpu.async_copy(rows_v.at[b], out_hbm.at[sid, j], sem_w)
      if j + RING < CPT:
        wbs[j].wait()
        cps[j + RING] = pltpu.async_copy(table_hbm.at[idx_v.at[j + RING]],
                                         rows_v.at[b], sem)
    for j in range(CPT - RING, CPT):
      wbs[j].wait()

  return gather_k


def _make_sc_scatter(d):
  mesh = plsc.VectorSubcoreMesh(core_axis_name="c", subcore_axis_name="s")

  @functools.partial(
      pl.kernel,
      mesh=mesh,
      out_type=jax.ShapeDtypeStruct((NC, ACC_CHUNKS, CHUNK, d), jnp.float32),
      scratch_types=[
          pltpu.VMEM((CPT, CHUNK), jnp.int32),
          pltpu.VMEM((2, CPB, CHUNK, d), jnp.float32),
          pltpu.VMEM_SHARED((HNPA, d), jnp.float32),
          pltpu.SemaphoreType.DMA,
          pltpu.SemaphoreType.DMA,
      ],
  )
  def scatter_k(msg_hbm, idx_hbm, zeros_hbm, out_hbm, idx_v, rows_v, acc_sh,
                sem, sem_a):
    cid = lax.axis_index("c")
    sid = lax.axis_index("s")
    pltpu.sync_copy(zeros_hbm, rows_v.at[0, 0])
    for i in range(3):
      g = sid * 3 + i
      @pl.when(g < ACC_CHUNKS)
      def _():
        pltpu.sync_copy(rows_v.at[0, 0], acc_sh.at[pl.ds(g * CHUNK, CHUNK)])
    pltpu.sync_copy(idx_hbm.at[cid, sid], idx_v)
    cp = pltpu.async_copy(msg_hbm.at[sid, 0], rows_v.at[0], sem)
    plsc.subcore_barrier()
    adds = {}
    for b in range(NBLK):
      cur = b % 2
      cp.wait()
      if b >= 1:
        for a in adds[b - 1]:
          a.wait()
      if b + 1 < NBLK:
        cp = pltpu.async_copy(msg_hbm.at[sid, b + 1], rows_v.at[1 - cur], sem)
      adds[b] = [
          pltpu.async_copy(rows_v.at[cur, j], acc_sh.at[idx_v.at[b * CPB + j]],
                           sem_a, add=True) for j in range(CPB)
      ]
    for a in adds[NBLK - 1]:
      a.wait()
    plsc.subcore_barrier()
    for i in range(3):
      g = sid * 3 + i
      @pl.when(g < ACC_CHUNKS)
      def _():
        pltpu.sync_copy(acc_sh.at[pl.ds(g * CHUNK, CHUNK)], rows_v.at[0, 0])
        pltpu.sync_copy(rows_v.at[0, 0], out_hbm.at[cid, g])

  return scatter_k


def _n_groups(dout):
  p = DP // dout
  return -(-(D_EDGE + 1) // p)


def _make_edge_msg(dout, din=DP, eb=512):
  grid = EP // eb
  ng = _n_groups(dout)
  ncol = ng * DP

  def body(ea_ref, xs_ref, w1_ref, b1_ref, w2_ref, msg_ref):
    hb = jnp.maximum(
        jnp.dot(ea_ref[...].astype(jnp.bfloat16), w1_ref[...],
                preferred_element_type=jnp.float32) + b1_ref[...], 0.0)
    y = jnp.dot(xs_ref[:, :din].astype(jnp.bfloat16), w2_ref[...],
                preferred_element_type=jnp.float32)
    acc = hb[:, :DP] * y[:, :DP]
    for g in range(1, ng):
      acc = acc + hb[:, g * DP:(g + 1) * DP] * y[:, g * DP:(g + 1) * DP]
    msg_ref[...] = acc

  def run(ea, xs, w1b, b1b, w2aug):
    return pl.pallas_call(
        body,
        grid=(grid,),
        in_specs=[
            pl.BlockSpec((eb, D_EDGE), lambda i: (i, 0)),
            pl.BlockSpec((eb, DP), lambda i: (i, 0)),
            pl.BlockSpec((D_EDGE, ncol), lambda i: (0, 0)),
            pl.BlockSpec((1, ncol), lambda i: (0, 0)),
            pl.BlockSpec((din, ncol), lambda i: (0, 0)),
        ],
        out_specs=pl.BlockSpec((eb, DP), lambda i: (i, 0)),
        out_shape=jax.ShapeDtypeStruct((EP, DP), jnp.float32),
    )(ea, xs, w1b, b1b, w2aug)

  return run


def _make_node_update(dout, nb=1000):
  grid = N // nb
  inv = 1.0 / dout

  def body(s_ref, x_ref, root_ref, bias_ref, g_ref, b_ref, out_ref):
    t = s_ref[...]
    w = DP
    while w > dout:
      w //= 2
      t = t[:, :w] + t[:, w:2 * w]
    z = (t + jnp.dot(x_ref[...], root_ref[...],
                     preferred_element_type=jnp.float32) + bias_ref[...])
    r = jnp.maximum(z, 0.0)
    mu = jnp.sum(r, axis=-1, keepdims=True) * inv
    sq = jnp.sum(r * r, axis=-1, keepdims=True) * inv
    var = sq - mu * mu
    out = (r - mu) * jax.lax.rsqrt(var + 1e-5) * g_ref[...] + b_ref[...]
    if dout < DP:
      out = jnp.concatenate(
          [out, jnp.zeros((nb, DP - dout), jnp.float32)], axis=1)
    out_ref[...] = out

  def run(s, x, root, biasr, gr, br):
    return pl.pallas_call(
        body,
        grid=(grid,),
        in_specs=[
            pl.BlockSpec((nb, DP), lambda i: (i, 0)),
            pl.BlockSpec((nb, DP), lambda i: (i, 0)),
            pl.BlockSpec((DP, dout), lambda i: (0, 0)),
            pl.BlockSpec((1, dout), lambda i: (0, 0)),
            pl.BlockSpec((1, dout), lambda i: (0, 0)),
            pl.BlockSpec((1, dout), lambda i: (0, 0)),
        ],
        out_specs=pl.BlockSpec((nb, DP), lambda i: (i, 0)),
        out_shape=jax.ShapeDtypeStruct((N, DP), jnp.float32),
    )(s, x, root, biasr, gr, br)

  return run


def _make_final(nb=1000):
  grid = N // nb

  def body(s_ref, x_ref, root_ref, bias_ref, w1_ref, b1_ref, w2_ref,
           b2_ref, emb_ref, logp_ref):
    emb = (s_ref[...]
           + jnp.dot(x_ref[...], root_ref[...],
                     preferred_element_type=jnp.float32) + bias_ref[...])
    emb_ref[...] = emb
    r = jnp.maximum(emb, 0.0)
    y = jnp.dot(r, w1_ref[...], preferred_element_type=jnp.float32) + b1_ref[...]
    y = jnp.dot(y, w2_ref[...], preferred_element_type=jnp.float32) + b2_ref[...]
    m = jnp.max(y, axis=-1, keepdims=True)
    lse = m + jnp.log(jnp.sum(jnp.exp(y - m), axis=-1, keepdims=True))
    logp_ref[...] = y - lse

  def run(s, x, root, biasr, w1, b1r, w2, b2r):
    return pl.pallas_call(
        body,
        grid=(grid,),
        in_specs=[
            pl.BlockSpec((nb, 128), lambda i: (i, 0)),
            pl.BlockSpec((nb, DP), lambda i: (i, 0)),
            pl.BlockSpec((DP, 128), lambda i: (0, 0)),
            pl.BlockSpec((1, 128), lambda i: (0, 0)),
            pl.BlockSpec((128, 64), lambda i: (0, 0)),
            pl.BlockSpec((1, 64), lambda i: (0, 0)),
            pl.BlockSpec((64, 32), lambda i: (0, 0)),
            pl.BlockSpec((1, 32), lambda i: (0, 0)),
        ],
        out_specs=[
            pl.BlockSpec((nb, 128), lambda i: (i, 0)),
            pl.BlockSpec((nb, 32), lambda i: (i, 0)),
        ],
        out_shape=[
            jax.ShapeDtypeStruct((N, 128), jnp.float32),
            jax.ShapeDtypeStruct((N, 32), jnp.float32),
        ],
    )(s, x, root, biasr, w1, b1r, w2, b2r)

  return run


def _w1aug(w1, b1, dout):
  p = DP // dout
  ng = _n_groups(dout)
  we = jnp.pad(jnp.pad(w1, ((0, 0), (0, 1))), ((0, 0), (0, ng * p - 17)))
  wb = jnp.broadcast_to(we[:, :, None], (D_EDGE, ng * p, dout))
  be = jnp.pad(jnp.concatenate([b1, jnp.ones((1,), jnp.float32)]),
               (0, ng * p - 17))
  bb = jnp.broadcast_to(be[:, None], (ng * p, dout))
  return (wb.reshape(D_EDGE, ng * DP).astype(jnp.bfloat16),
          bb.reshape(1, ng * DP))


def _w2aug(w2, b2, din, dout):
  p = DP // dout
  ng = _n_groups(dout)
  w = w2.reshape(D_EDGE, din, dout).transpose(1, 0, 2)
  w = jnp.concatenate([w, b2.reshape(din, 1, dout)], axis=1)
  w = jnp.pad(w, ((0, 0), (0, ng * p - 17), (0, 0)))
  return w.reshape(din, ng * DP).astype(jnp.bfloat16)


def _pad_cols(v, width=DP):
  return jnp.pad(v, ((0, 0), (0, width - v.shape[1])))


def _pad_vec(v, width=DP):
  return jnp.pad(v.reshape(1, -1), ((0, 0), (0, width - v.shape[0])))


def kernel(x, edge_index, edge_attr,
           en1_W1, en1_b1, en1_W2, en1_b2, root1, bias1, ln1_g, ln1_b,
           en2_W1, en2_b1, en2_W2, en2_b2, root2, bias2, ln2_g, ln2_b,
           en3_W1, en3_b1, en3_W2, en3_b2, root3, bias3,
           pm_W1, pm_b1, pm_W2, pm_b2):
  pad = EP - E
  src = jnp.concatenate(
      [edge_index[0].astype(jnp.int32),
       jnp.zeros((pad,), jnp.int32)]).reshape(NS, CPT, CHUNK)
  dst = jnp.concatenate(
      [edge_index[1].astype(jnp.int32),
       jnp.full((pad,), N, jnp.int32)])
  trash = HNP + (jnp.arange(EP, dtype=jnp.int32) % CHUNK)
  dstc = jnp.stack([
      jnp.where((dst >= c * HNP) & (dst < (c + 1) * HNP), dst - c * HNP,
                trash) for c in range(NC)
  ]).reshape(NC, NS, CPT, CHUNK)
  ea = jnp.concatenate([edge_attr, jnp.zeros((pad, D_EDGE), jnp.float32)])

  gather = _make_sc_gather()

  def r2(v):
    return v.reshape(1, -1)

  scatter = _make_sc_scatter(DP)
  zeros = jnp.zeros((CHUNK, DP), jnp.float32)

  def agg(msg):
    s = scatter(msg.reshape(NS, NBLK, CPB, CHUNK, DP), dstc, zeros)
    return s[:, :HNP // CHUNK].reshape(NP, DP)[:N]

  xp = _pad_cols(x)

  xs = gather(xp, src).reshape(EP, DP)
  msg = _make_edge_msg(32, 64)(ea, xs, *_w1aug(en1_W1, en1_b1, 32),
                               _w2aug(en1_W2, en1_b2, 64, 32))
  root1p = jnp.pad(root1, ((0, DP - 64), (0, 0)))
  h1 = _make_node_update(32)(agg(msg), xp, root1p, r2(bias1),
                             r2(ln1_g), r2(ln1_b))

  xs = gather(h1, src).reshape(EP, DP)
  msg = _make_edge_msg(64, 32)(ea, xs, *_w1aug(en2_W1, en2_b1, 64),
                               _w2aug(en2_W2, en2_b2, 32, 64))
  root2p = jnp.pad(root2, ((0, DP - 32), (0, 0)))
  h2 = _make_node_update(64)(agg(msg), h1, root2p, r2(bias2),
                             r2(ln2_g), r2(ln2_b))

  xs = gather(h2, src).reshape(EP, DP)
  msg = _make_edge_msg(128, 64)(ea, xs, *_w1aug(en3_W1, en3_b1, 128),
                                _w2aug(en3_W2, en3_b2, 64, 128))
  root3p = jnp.pad(root3, ((0, DP - 64), (0, 0)))
  emb, logp = _make_final()(agg(msg), h2, root3p, r2(bias3),
                            pm_W1, r2(pm_b1), pm_W2, r2(pm_b2))
  return (emb, logp)


# speedup vs baseline: 1.1254x; 1.0613x over previous
import functools

import jax
import jax.numpy as jnp
from jax import lax
from jax.experimental import pallas as pl
from jax.experimental.pallas import tpu as pltpu
from jax.experimental.pallas import tpu_sc as plsc

N = 10000
E = 20000
D_EDGE = 16
DP = 128

NC = 2
NS = 16
NW = NC * NS
CHUNK = 128
NCHUNK = 5
EB_TILE = NCHUNK * CHUNK
EP = NW * EB_TILE
NP = 10240
ROWS_TILE = NP // NS
HNP = NP // NC
ACC_CHUNKS = HNP // CHUNK + 1
HNPA = ACC_CHUNKS * CHUNK
NCHUNKS_ALL = EP // CHUNK
CPT = NCHUNKS_ALL // NS
CPB = 2
NBLK = CPT // CPB


RING = 6


def _make_sc_gather():
  mesh = plsc.VectorSubcoreMesh(core_axis_name="c", subcore_axis_name="s",
                                num_cores=1)

  @functools.partial(
      pl.kernel,
      mesh=mesh,
      out_type=jax.ShapeDtypeStruct((NS, CPT, CHUNK, DP), jnp.float32),
      scratch_types=[
          pltpu.VMEM((CPT, CHUNK), jnp.int32),
          pltpu.VMEM((RING, CHUNK, DP), jnp.float32),
          pltpu.SemaphoreType.DMA,
          pltpu.SemaphoreType.DMA,
      ],
  )
  def gather_k(table_hbm, idx_hbm, out_hbm, idx_v, rows_v, sem, sem_w):
    sid = lax.axis_index("s")
    pltpu.sync_copy(idx_hbm.at[sid], idx_v)
    cps = {}
    wbs = {}
    for j in range(RING):
      cps[j] = pltpu.async_copy(table_hbm.at[idx_v.at[j]], rows_v.at[j], sem)
    for j in range(CPT):
      b = j % RING
      cps[j].wait()
      wbs[j] = pltpu.async_copy(rows_v.at[b], out_hbm.at[sid, j], sem_w)
      if j + RING < CPT:
        wbs[j].wait()
        cps[j + RING] = pltpu.async_copy(table_hbm.at[idx_v.at[j + RING]],
                                         rows_v.at[b], sem)
    for j in range(CPT - RING, CPT):
      wbs[j].wait()

  return gather_k


def _make_sc_scatter(d):
  mesh = plsc.VectorSubcoreMesh(core_axis_name="c", subcore_axis_name="s")

  @functools.partial(
      pl.kernel,
      mesh=mesh,
      out_type=jax.ShapeDtypeStruct((NC, ACC_CHUNKS, CHUNK, d), jnp.float32),
      scratch_types=[
          pltpu.VMEM((CPT, CHUNK), jnp.int32),
          pltpu.VMEM((2, CPB, CHUNK, d), jnp.float32),
          pltpu.VMEM_SHARED((HNPA, d), jnp.float32),
          pltpu.SemaphoreType.DMA,
          pltpu.SemaphoreType.DMA,
      ],
  )
  def scatter_k(msg_hbm, idx_hbm, zeros_hbm, out_hbm, idx_v, rows_v, acc_sh,
                sem, sem_a):
    cid = lax.axis_index("c")
    sid = lax.axis_index("s")
    pltpu.sync_copy(zeros_hbm, rows_v.at[0, 0])
    for i in range(3):
      g = sid * 3 + i
      @pl.when(g < ACC_CHUNKS)
      def _():
        pltpu.sync_copy(rows_v.at[0, 0], acc_sh.at[pl.ds(g * CHUNK, CHUNK)])
    pltpu.sync_copy(idx_hbm.at[cid, sid], idx_v)
    cp = pltpu.async_copy(msg_hbm.at[sid, 0], rows_v.at[0], sem)
    plsc.subcore_barrier()
    adds = {}
    for b in range(NBLK):
      cur = b % 2
      cp.wait()
      if b >= 1:
        for a in adds[b - 1]:
          a.wait()
      if b + 1 < NBLK:
        cp = pltpu.async_copy(msg_hbm.at[sid, b + 1], rows_v.at[1 - cur], sem)
      adds[b] = [
          pltpu.async_copy(rows_v.at[cur, j], acc_sh.at[idx_v.at[b * CPB + j]],
                           sem_a, add=True) for j in range(CPB)
      ]
    for a in adds[NBLK - 1]:
      a.wait()
    plsc.subcore_barrier()
    for i in range(3):
      g = sid * 3 + i
      @pl.when(g < ACC_CHUNKS)
      def _():
        pltpu.sync_copy(acc_sh.at[pl.ds(g * CHUNK, CHUNK)], rows_v.at[0, 0])
        pltpu.sync_copy(rows_v.at[0, 0], out_hbm.at[cid, g])

  return scatter_k


def _n_groups(dout):
  p = DP // dout
  return -(-(D_EDGE + 1) // p)


def _make_edge_msg(dout, din=DP, eb=1024):
  grid = EP // eb
  ng = _n_groups(dout)
  ncol = ng * DP

  def body(ea_ref, xs_ref, w1_ref, b1_ref, w2_ref, msg_ref):
    hb = jnp.maximum(
        jnp.dot(ea_ref[...].astype(jnp.bfloat16), w1_ref[...],
                preferred_element_type=jnp.float32) + b1_ref[...], 0.0)
    y = jnp.dot(xs_ref[:, :din].astype(jnp.bfloat16), w2_ref[...],
                preferred_element_type=jnp.float32)
    acc = hb[:, :DP] * y[:, :DP]
    for g in range(1, ng):
      acc = acc + hb[:, g * DP:(g + 1) * DP] * y[:, g * DP:(g + 1) * DP]
    msg_ref[...] = acc

  def run(ea, xs, w1b, b1b, w2aug):
    return pl.pallas_call(
        body,
        grid=(grid,),
        in_specs=[
            pl.BlockSpec((eb, D_EDGE), lambda i: (i, 0)),
            pl.BlockSpec((eb, DP), lambda i: (i, 0)),
            pl.BlockSpec((D_EDGE, ncol), lambda i: (0, 0)),
            pl.BlockSpec((1, ncol), lambda i: (0, 0)),
            pl.BlockSpec((din, ncol), lambda i: (0, 0)),
        ],
        out_specs=pl.BlockSpec((eb, DP), lambda i: (i, 0)),
        out_shape=jax.ShapeDtypeStruct((EP, DP), jnp.float32),
    )(ea, xs, w1b, b1b, w2aug)

  return run


def _make_node_update(dout, nb=1000):
  grid = N // nb
  inv = 1.0 / dout

  def body(s_ref, x_ref, root_ref, bias_ref, g_ref, b_ref, out_ref):
    t = s_ref[...]
    w = DP
    while w > dout:
      w //= 2
      t = t[:, :w] + t[:, w:2 * w]
    z = (t + jnp.dot(x_ref[...], root_ref[...],
                     preferred_element_type=jnp.float32) + bias_ref[...])
    r = jnp.maximum(z, 0.0)
    mu = jnp.sum(r, axis=-1, keepdims=True) * inv
    sq = jnp.sum(r * r, axis=-1, keepdims=True) * inv
    var = sq - mu * mu
    out = (r - mu) * jax.lax.rsqrt(var + 1e-5) * g_ref[...] + b_ref[...]
    if dout < DP:
      out = jnp.concatenate(
          [out, jnp.zeros((nb, DP - dout), jnp.float32)], axis=1)
    out_ref[...] = out

  def run(s, x, root, biasr, gr, br):
    return pl.pallas_call(
        body,
        grid=(grid,),
        in_specs=[
            pl.BlockSpec((nb, DP), lambda i: (i, 0)),
            pl.BlockSpec((nb, DP), lambda i: (i, 0)),
            pl.BlockSpec((DP, dout), lambda i: (0, 0)),
            pl.BlockSpec((1, dout), lambda i: (0, 0)),
            pl.BlockSpec((1, dout), lambda i: (0, 0)),
            pl.BlockSpec((1, dout), lambda i: (0, 0)),
        ],
        out_specs=pl.BlockSpec((nb, DP), lambda i: (i, 0)),
        out_shape=jax.ShapeDtypeStruct((N, DP), jnp.float32),
    )(s, x, root, biasr, gr, br)

  return run


def _make_final(nb=1000):
  grid = N // nb

  def body(s_ref, x_ref, root_ref, bias_ref, w1_ref, b1_ref, w2_ref,
           b2_ref, emb_ref, logp_ref):
    emb = (s_ref[...]
           + jnp.dot(x_ref[...], root_ref[...],
                     preferred_element_type=jnp.float32) + bias_ref[...])
    emb_ref[...] = emb
    r = jnp.maximum(emb, 0.0)
    y = jnp.dot(r, w1_ref[...], preferred_element_type=jnp.float32) + b1_ref[...]
    y = jnp.dot(y, w2_ref[...], preferred_element_type=jnp.float32) + b2_ref[...]
    m = jnp.max(y, axis=-1, keepdims=True)
    lse = m + jnp.log(jnp.sum(jnp.exp(y - m), axis=-1, keepdims=True))
    logp_ref[...] = y - lse

  def run(s, x, root, biasr, w1, b1r, w2, b2r):
    return pl.pallas_call(
        body,
        grid=(grid,),
        in_specs=[
            pl.BlockSpec((nb, 128), lambda i: (i, 0)),
            pl.BlockSpec((nb, DP), lambda i: (i, 0)),
            pl.BlockSpec((DP, 128), lambda i: (0, 0)),
            pl.BlockSpec((1, 128), lambda i: (0, 0)),
            pl.BlockSpec((128, 64), lambda i: (0, 0)),
            pl.BlockSpec((1, 64), lambda i: (0, 0)),
            pl.BlockSpec((64, 32), lambda i: (0, 0)),
            pl.BlockSpec((1, 32), lambda i: (0, 0)),
        ],
        out_specs=[
            pl.BlockSpec((nb, 128), lambda i: (i, 0)),
            pl.BlockSpec((nb, 32), lambda i: (i, 0)),
        ],
        out_shape=[
            jax.ShapeDtypeStruct((N, 128), jnp.float32),
            jax.ShapeDtypeStruct((N, 32), jnp.float32),
        ],
    )(s, x, root, biasr, w1, b1r, w2, b2r)

  return run


def _w1aug(w1, b1, dout):
  p = DP // dout
  ng = _n_groups(dout)
  we = jnp.pad(jnp.pad(w1, ((0, 0), (0, 1))), ((0, 0), (0, ng * p - 17)))
  wb = jnp.broadcast_to(we[:, :, None], (D_EDGE, ng * p, dout))
  be = jnp.pad(jnp.concatenate([b1, jnp.ones((1,), jnp.float32)]),
               (0, ng * p - 17))
  bb = jnp.broadcast_to(be[:, None], (ng * p, dout))
  return (wb.reshape(D_EDGE, ng * DP).astype(jnp.bfloat16),
          bb.reshape(1, ng * DP))


def _w2aug(w2, b2, din, dout):
  p = DP // dout
  ng = _n_groups(dout)
  w = w2.reshape(D_EDGE, din, dout).transpose(1, 0, 2)
  w = jnp.concatenate([w, b2.reshape(din, 1, dout)], axis=1)
  w = jnp.pad(w, ((0, 0), (0, ng * p - 17), (0, 0)))
  return w.reshape(din, ng * DP).astype(jnp.bfloat16)


def _pad_cols(v, width=DP):
  return jnp.pad(v, ((0, 0), (0, width - v.shape[1])))


def _pad_vec(v, width=DP):
  return jnp.pad(v.reshape(1, -1), ((0, 0), (0, width - v.shape[0])))


def kernel(x, edge_index, edge_attr,
           en1_W1, en1_b1, en1_W2, en1_b2, root1, bias1, ln1_g, ln1_b,
           en2_W1, en2_b1, en2_W2, en2_b2, root2, bias2, ln2_g, ln2_b,
           en3_W1, en3_b1, en3_W2, en3_b2, root3, bias3,
           pm_W1, pm_b1, pm_W2, pm_b2):
  pad = EP - E
  src = jnp.concatenate(
      [edge_index[0].astype(jnp.int32),
       jnp.zeros((pad,), jnp.int32)]).reshape(NS, CPT, CHUNK)
  dst = jnp.concatenate(
      [edge_index[1].astype(jnp.int32),
       jnp.full((pad,), N, jnp.int32)])
  trash = HNP + (jnp.arange(EP, dtype=jnp.int32) % CHUNK)
  dstc = jnp.stack([
      jnp.where((dst >= c * HNP) & (dst < (c + 1) * HNP), dst - c * HNP,
                trash) for c in range(NC)
  ]).reshape(NC, NS, CPT, CHUNK)
  ea = jnp.concatenate([edge_attr, jnp.zeros((pad, D_EDGE), jnp.float32)])

  gather = _make_sc_gather()

  def r2(v):
    return v.reshape(1, -1)

  scatter = _make_sc_scatter(DP)
  zeros = jnp.zeros((CHUNK, DP), jnp.float32)

  def agg(msg):
    s = scatter(msg.reshape(NS, NBLK, CPB, CHUNK, DP), dstc, zeros)
    return s[:, :HNP // CHUNK].reshape(NP, DP)[:N]

  xp = _pad_cols(x)

  xs = gather(xp, src).reshape(EP, DP)
  msg = _make_edge_msg(32, 64)(ea, xs, *_w1aug(en1_W1, en1_b1, 32),
                               _w2aug(en1_W2, en1_b2, 64, 32))
  root1p = jnp.pad(root1, ((0, DP - 64), (0, 0)))
  h1 = _make_node_update(32)(agg(msg), xp, root1p, r2(bias1),
                             r2(ln1_g), r2(ln1_b))

  xs = gather(h1, src).reshape(EP, DP)
  msg = _make_edge_msg(64, 32)(ea, xs, *_w1aug(en2_W1, en2_b1, 64),
                               _w2aug(en2_W2, en2_b2, 32, 64))
  root2p = jnp.pad(root2, ((0, DP - 32), (0, 0)))
  h2 = _make_node_update(64)(agg(msg), h1, root2p, r2(bias2),
                             r2(ln2_g), r2(ln2_b))

  xs = gather(h2, src).reshape(EP, DP)
  msg = _make_edge_msg(128, 64)(ea, xs, *_w1aug(en3_W1, en3_b1, 128),
                                _w2aug(en3_W2, en3_b2, 64, 128))
  root3p = jnp.pad(root3, ((0, DP - 64), (0, 0)))
  emb, logp = _make_final()(agg(msg), h2, root3p, r2(bias3),
                            pm_W1, r2(pm_b1), pm_W2, r2(pm_b2))
  return (emb, logp)


# speedup vs baseline: 1.1517x; 1.0234x over previous
import functools

import jax
import jax.numpy as jnp
from jax import lax
from jax.experimental import pallas as pl
from jax.experimental.pallas import tpu as pltpu
from jax.experimental.pallas import tpu_sc as plsc

N = 10000
E = 20000
D_EDGE = 16
DP = 128

NC = 2
NS = 16
NW = NC * NS
CHUNK = 128
NCHUNK = 5
EB_TILE = NCHUNK * CHUNK
EP = NW * EB_TILE
NP = 10240
ROWS_TILE = NP // NS
HNP = NP // NC
ACC_CHUNKS = HNP // CHUNK + 1
HNPA = ACC_CHUNKS * CHUNK
NCHUNKS_ALL = EP // CHUNK
CPT = NCHUNKS_ALL // NS
CPB = 2
NBLK = CPT // CPB


RING = 6


def _make_sc_gather():
  mesh = plsc.VectorSubcoreMesh(core_axis_name="c", subcore_axis_name="s",
                                num_cores=1)

  @functools.partial(
      pl.kernel,
      mesh=mesh,
      out_type=jax.ShapeDtypeStruct((NS, CPT, CHUNK, DP), jnp.float32),
      scratch_types=[
          pltpu.VMEM((CPT, CHUNK), jnp.int32),
          pltpu.VMEM((RING, CHUNK, DP), jnp.float32),
          pltpu.SemaphoreType.DMA,
          pltpu.SemaphoreType.DMA,
      ],
  )
  def gather_k(table_hbm, idx_hbm, out_hbm, idx_v, rows_v, sem, sem_w):
    sid = lax.axis_index("s")
    pltpu.sync_copy(idx_hbm.at[sid], idx_v)
    cps = {}
    wbs = {}
    for j in range(RING):
      cps[j] = pltpu.async_copy(table_hbm.at[idx_v.at[j]], rows_v.at[j], sem)
    for j in range(CPT):
      b = j % RING
      cps[j].wait()
      wbs[j] = pltpu.async_copy(rows_v.at[b], out_hbm.at[sid, j], sem_w)
      if j + RING < CPT:
        wbs[j].wait()
        cps[j + RING] = pltpu.async_copy(table_hbm.at[idx_v.at[j + RING]],
                                         rows_v.at[b], sem)
    for j in range(CPT - RING, CPT):
      wbs[j].wait()

  return gather_k


def _make_sc_scatter(d):
  mesh = plsc.VectorSubcoreMesh(core_axis_name="c", subcore_axis_name="s")

  @functools.partial(
      pl.kernel,
      mesh=mesh,
      out_type=jax.ShapeDtypeStruct((NC, ACC_CHUNKS, CHUNK, d), jnp.float32),
      scratch_types=[
          pltpu.VMEM((CPT, CHUNK), jnp.int32),
          pltpu.VMEM((2, CPB, CHUNK, d), jnp.float32),
          pltpu.VMEM_SHARED((HNPA, d), jnp.float32),
          pltpu.SemaphoreType.DMA,
          pltpu.SemaphoreType.DMA,
      ],
  )
  def scatter_k(msg_hbm, idx_hbm, zeros_hbm, out_hbm, idx_v, rows_v, acc_sh,
                sem, sem_a):
    cid = lax.axis_index("c")
    sid = lax.axis_index("s")
    pltpu.sync_copy(zeros_hbm, rows_v.at[0, 0])
    for i in range(3):
      g = sid * 3 + i
      @pl.when(g < ACC_CHUNKS)
      def _():
        pltpu.sync_copy(rows_v.at[0, 0], acc_sh.at[pl.ds(g * CHUNK, CHUNK)])
    pltpu.sync_copy(idx_hbm.at[cid, sid], idx_v)
    cp = pltpu.async_copy(msg_hbm.at[sid, 0], rows_v.at[0], sem)
    plsc.subcore_barrier()
    adds = {}
    for b in range(NBLK):
      cur = b % 2
      cp.wait()
      if b >= 1:
        for a in adds[b - 1]:
          a.wait()
      if b + 1 < NBLK:
        cp = pltpu.async_copy(msg_hbm.at[sid, b + 1], rows_v.at[1 - cur], sem)
      adds[b] = [
          pltpu.async_copy(rows_v.at[cur, j], acc_sh.at[idx_v.at[b * CPB + j]],
                           sem_a, add=True) for j in range(CPB)
      ]
    for a in adds[NBLK - 1]:
      a.wait()
    plsc.subcore_barrier()
    for i in range(3):
      g = sid * 3 + i
      @pl.when(g < ACC_CHUNKS)
      def _():
        pltpu.sync_copy(acc_sh.at[pl.ds(g * CHUNK, CHUNK)], rows_v.at[0, 0])
        pltpu.sync_copy(rows_v.at[0, 0], out_hbm.at[cid, g])

  return scatter_k


def _n_groups(dout):
  p = DP // dout
  return -(-(D_EDGE + 1) // p)


def _make_edge_msg(dout, din=DP, eb=2048):
  grid = EP // eb
  ng = _n_groups(dout)
  ncol = ng * DP

  def body(ea_ref, xs_ref, w1_ref, b1_ref, w2_ref, msg_ref):
    hb = jnp.maximum(
        jnp.dot(ea_ref[...].astype(jnp.bfloat16), w1_ref[...],
                preferred_element_type=jnp.float32) + b1_ref[...], 0.0)
    y = jnp.dot(xs_ref[:, :din].astype(jnp.bfloat16), w2_ref[...],
                preferred_element_type=jnp.float32)
    acc = hb[:, :DP] * y[:, :DP]
    for g in range(1, ng):
      acc = acc + hb[:, g * DP:(g + 1) * DP] * y[:, g * DP:(g + 1) * DP]
    msg_ref[...] = acc

  def run(ea, xs, w1b, b1b, w2aug):
    return pl.pallas_call(
        body,
        grid=(grid,),
        in_specs=[
            pl.BlockSpec((eb, D_EDGE), lambda i: (i, 0)),
            pl.BlockSpec((eb, DP), lambda i: (i, 0)),
            pl.BlockSpec((D_EDGE, ncol), lambda i: (0, 0)),
            pl.BlockSpec((1, ncol), lambda i: (0, 0)),
            pl.BlockSpec((din, ncol), lambda i: (0, 0)),
        ],
        out_specs=pl.BlockSpec((eb, DP), lambda i: (i, 0)),
        out_shape=jax.ShapeDtypeStruct((EP, DP), jnp.float32),
    )(ea, xs, w1b, b1b, w2aug)

  return run


def _make_node_update(dout, nb=1000):
  grid = N // nb
  inv = 1.0 / dout

  def body(s_ref, x_ref, root_ref, bias_ref, g_ref, b_ref, out_ref):
    t = s_ref[...]
    w = DP
    while w > dout:
      w //= 2
      t = t[:, :w] + t[:, w:2 * w]
    z = (t + jnp.dot(x_ref[...], root_ref[...],
                     preferred_element_type=jnp.float32) + bias_ref[...])
    r = jnp.maximum(z, 0.0)
    mu = jnp.sum(r, axis=-1, keepdims=True) * inv
    sq = jnp.sum(r * r, axis=-1, keepdims=True) * inv
    var = sq - mu * mu
    out = (r - mu) * jax.lax.rsqrt(var + 1e-5) * g_ref[...] + b_ref[...]
    if dout < DP:
      out = jnp.concatenate(
          [out, jnp.zeros((nb, DP - dout), jnp.float32)], axis=1)
    out_ref[...] = out

  def run(s, x, root, biasr, gr, br):
    return pl.pallas_call(
        body,
        grid=(grid,),
        in_specs=[
            pl.BlockSpec((nb, DP), lambda i: (i, 0)),
            pl.BlockSpec((nb, DP), lambda i: (i, 0)),
            pl.BlockSpec((DP, dout), lambda i: (0, 0)),
            pl.BlockSpec((1, dout), lambda i: (0, 0)),
            pl.BlockSpec((1, dout), lambda i: (0, 0)),
            pl.BlockSpec((1, dout), lambda i: (0, 0)),
        ],
        out_specs=pl.BlockSpec((nb, DP), lambda i: (i, 0)),
        out_shape=jax.ShapeDtypeStruct((N, DP), jnp.float32),
    )(s, x, root, biasr, gr, br)

  return run


def _make_final(nb=1000):
  grid = N // nb

  def body(s_ref, x_ref, root_ref, bias_ref, w1_ref, b1_ref, w2_ref,
           b2_ref, emb_ref, logp_ref):
    emb = (s_ref[...]
           + jnp.dot(x_ref[...], root_ref[...],
                     preferred_element_type=jnp.float32) + bias_ref[...])
    emb_ref[...] = emb
    r = jnp.maximum(emb, 0.0)
    y = jnp.dot(r, w1_ref[...], preferred_element_type=jnp.float32) + b1_ref[...]
    y = jnp.dot(y, w2_ref[...], preferred_element_type=jnp.float32) + b2_ref[...]
    m = jnp.max(y, axis=-1, keepdims=True)
    lse = m + jnp.log(jnp.sum(jnp.exp(y - m), axis=-1, keepdims=True))
    logp_ref[...] = y - lse

  def run(s, x, root, biasr, w1, b1r, w2, b2r):
    return pl.pallas_call(
        body,
        grid=(grid,),
        in_specs=[
            pl.BlockSpec((nb, 128), lambda i: (i, 0)),
            pl.BlockSpec((nb, DP), lambda i: (i, 0)),
            pl.BlockSpec((DP, 128), lambda i: (0, 0)),
            pl.BlockSpec((1, 128), lambda i: (0, 0)),
            pl.BlockSpec((128, 64), lambda i: (0, 0)),
            pl.BlockSpec((1, 64), lambda i: (0, 0)),
            pl.BlockSpec((64, 32), lambda i: (0, 0)),
            pl.BlockSpec((1, 32), lambda i: (0, 0)),
        ],
        out_specs=[
            pl.BlockSpec((nb, 128), lambda i: (i, 0)),
            pl.BlockSpec((nb, 32), lambda i: (i, 0)),
        ],
        out_shape=[
            jax.ShapeDtypeStruct((N, 128), jnp.float32),
            jax.ShapeDtypeStruct((N, 32), jnp.float32),
        ],
    )(s, x, root, biasr, w1, b1r, w2, b2r)

  return run


def _w1aug(w1, b1, dout):
  p = DP // dout
  ng = _n_groups(dout)
  we = jnp.pad(jnp.pad(w1, ((0, 0), (0, 1))), ((0, 0), (0, ng * p - 17)))
  wb = jnp.broadcast_to(we[:, :, None], (D_EDGE, ng * p, dout))
  be = jnp.pad(jnp.concatenate([b1, jnp.ones((1,), jnp.float32)]),
               (0, ng * p - 17))
  bb = jnp.broadcast_to(be[:, None], (ng * p, dout))
  return (wb.reshape(D_EDGE, ng * DP).astype(jnp.bfloat16),
          bb.reshape(1, ng * DP))


def _w2aug(w2, b2, din, dout):
  p = DP // dout
  ng = _n_groups(dout)
  w = w2.reshape(D_EDGE, din, dout).transpose(1, 0, 2)
  w = jnp.concatenate([w, b2.reshape(din, 1, dout)], axis=1)
  w = jnp.pad(w, ((0, 0), (0, ng * p - 17), (0, 0)))
  return w.reshape(din, ng * DP).astype(jnp.bfloat16)


def _pad_cols(v, width=DP):
  return jnp.pad(v, ((0, 0), (0, width - v.shape[1])))


def _pad_vec(v, width=DP):
  return jnp.pad(v.reshape(1, -1), ((0, 0), (0, width - v.shape[0])))


def kernel(x, edge_index, edge_attr,
           en1_W1, en1_b1, en1_W2, en1_b2, root1, bias1, ln1_g, ln1_b,
           en2_W1, en2_b1, en2_W2, en2_b2, root2, bias2, ln2_g, ln2_b,
           en3_W1, en3_b1, en3_W2, en3_b2, root3, bias3,
           pm_W1, pm_b1, pm_W2, pm_b2):
  pad = EP - E
  src = jnp.concatenate(
      [edge_index[0].astype(jnp.int32),
       jnp.zeros((pad,), jnp.int32)]).reshape(NS, CPT, CHUNK)
  dst = jnp.concatenate(
      [edge_index[1].astype(jnp.int32),
       jnp.full((pad,), N, jnp.int32)])
  trash = HNP + (jnp.arange(EP, dtype=jnp.int32) % CHUNK)
  dstc = jnp.stack([
      jnp.where((dst >= c * HNP) & (dst < (c + 1) * HNP), dst - c * HNP,
                trash) for c in range(NC)
  ]).reshape(NC, NS, CPT, CHUNK)
  ea = jnp.concatenate([edge_attr, jnp.zeros((pad, D_EDGE), jnp.float32)])

  gather = _make_sc_gather()

  def r2(v):
    return v.reshape(1, -1)

  scatter = _make_sc_scatter(DP)
  zeros = jnp.zeros((CHUNK, DP), jnp.float32)

  def agg(msg):
    s = scatter(msg.reshape(NS, NBLK, CPB, CHUNK, DP), dstc, zeros)
    return s[:, :HNP // CHUNK].reshape(NP, DP)[:N]

  xp = _pad_cols(x)

  xs = gather(xp, src).reshape(EP, DP)
  msg = _make_edge_msg(32, 64)(ea, xs, *_w1aug(en1_W1, en1_b1, 32),
                               _w2aug(en1_W2, en1_b2, 64, 32))
  root1p = jnp.pad(root1, ((0, DP - 64), (0, 0)))
  h1 = _make_node_update(32)(agg(msg), xp, root1p, r2(bias1),
                             r2(ln1_g), r2(ln1_b))

  xs = gather(h1, src).reshape(EP, DP)
  msg = _make_edge_msg(64, 32)(ea, xs, *_w1aug(en2_W1, en2_b1, 64),
                               _w2aug(en2_W2, en2_b2, 32, 64))
  root2p = jnp.pad(root2, ((0, DP - 32), (0, 0)))
  h2 = _make_node_update(64)(agg(msg), h1, root2p, r2(bias2),
                             r2(ln2_g), r2(ln2_b))

  xs = gather(h2, src).reshape(EP, DP)
  msg = _make_edge_msg(128, 64)(ea, xs, *_w1aug(en3_W1, en3_b1, 128),
                                _w2aug(en3_W2, en3_b2, 64, 128))
  root3p = jnp.pad(root3, ((0, DP - 64), (0, 0)))
  emb, logp = _make_final()(agg(msg), h2, root3p, r2(bias3),
                            pm_W1, r2(pm_b1), pm_W2, r2(pm_b2))
  return (emb, logp)


# speedup vs baseline: 1.1589x; 1.0063x over previous
import functools

import jax
import jax.numpy as jnp
from jax import lax
from jax.experimental import pallas as pl
from jax.experimental.pallas import tpu as pltpu
from jax.experimental.pallas import tpu_sc as plsc

N = 10000
E = 20000
D_EDGE = 16
DP = 128

NC = 2
NS = 16
NW = NC * NS
CHUNK = 128
NCHUNK = 5
EB_TILE = NCHUNK * CHUNK
EP = NW * EB_TILE
NP = 10240
ROWS_TILE = NP // NS
HNP = NP // NC
ACC_CHUNKS = HNP // CHUNK + 1
HNPA = ACC_CHUNKS * CHUNK
NCHUNKS_ALL = EP // CHUNK
CPT = NCHUNKS_ALL // NS
CPB = 2
NBLK = CPT // CPB


RING = 6


def _make_sc_gather():
  mesh = plsc.VectorSubcoreMesh(core_axis_name="c", subcore_axis_name="s",
                                num_cores=1)

  @functools.partial(
      pl.kernel,
      mesh=mesh,
      out_type=jax.ShapeDtypeStruct((NS, CPT, CHUNK, DP), jnp.float32),
      scratch_types=[
          pltpu.VMEM((CPT, CHUNK), jnp.int32),
          pltpu.VMEM((RING, CHUNK, DP), jnp.float32),
          pltpu.SemaphoreType.DMA,
          pltpu.SemaphoreType.DMA,
      ],
  )
  def gather_k(table_hbm, idx_hbm, out_hbm, idx_v, rows_v, sem, sem_w):
    sid = lax.axis_index("s")
    pltpu.sync_copy(idx_hbm.at[sid], idx_v)
    cps = {}
    wbs = {}
    for j in range(RING):
      cps[j] = pltpu.async_copy(table_hbm.at[idx_v.at[j]], rows_v.at[j], sem)
    for j in range(CPT):
      b = j % RING
      cps[j].wait()
      wbs[j] = pltpu.async_copy(rows_v.at[b], out_hbm.at[sid, j], sem_w)
      if j + RING < CPT:
        wbs[j].wait()
        cps[j + RING] = pltpu.async_copy(table_hbm.at[idx_v.at[j + RING]],
                                         rows_v.at[b], sem)
    for j in range(CPT - RING, CPT):
      wbs[j].wait()

  return gather_k


def _make_sc_scatter(d):
  mesh = plsc.VectorSubcoreMesh(core_axis_name="c", subcore_axis_name="s")

  @functools.partial(
      pl.kernel,
      mesh=mesh,
      out_type=jax.ShapeDtypeStruct((NC, ACC_CHUNKS, CHUNK, d), jnp.float32),
      scratch_types=[
          pltpu.VMEM((CPT, CHUNK), jnp.int32),
          pltpu.VMEM((2, CPB, CHUNK, d), jnp.float32),
          pltpu.VMEM_SHARED((HNPA, d), jnp.float32),
          pltpu.SemaphoreType.DMA,
          pltpu.SemaphoreType.DMA,
      ],
  )
  def scatter_k(msg_hbm, idx_hbm, zeros_hbm, out_hbm, idx_v, rows_v, acc_sh,
                sem, sem_a):
    cid = lax.axis_index("c")
    sid = lax.axis_index("s")
    pltpu.sync_copy(zeros_hbm, rows_v.at[0, 0])
    for i in range(3):
      g = sid * 3 + i
      @pl.when(g < ACC_CHUNKS)
      def _():
        pltpu.sync_copy(rows_v.at[0, 0], acc_sh.at[pl.ds(g * CHUNK, CHUNK)])
    pltpu.sync_copy(idx_hbm.at[cid, sid], idx_v)
    cp = pltpu.async_copy(msg_hbm.at[sid, 0], rows_v.at[0], sem)
    plsc.subcore_barrier()
    adds = {}
    for b in range(NBLK):
      cur = b % 2
      cp.wait()
      if b >= 1:
        for a in adds[b - 1]:
          a.wait()
      if b + 1 < NBLK:
        cp = pltpu.async_copy(msg_hbm.at[sid, b + 1], rows_v.at[1 - cur], sem)
      adds[b] = [
          pltpu.async_copy(rows_v.at[cur, j], acc_sh.at[idx_v.at[b * CPB + j]],
                           sem_a, add=True) for j in range(CPB)
      ]
    for a in adds[NBLK - 1]:
      a.wait()
    plsc.subcore_barrier()
    for i in range(3):
      g = sid * 3 + i
      @pl.when(g < ACC_CHUNKS)
      def _():
        pltpu.sync_copy(acc_sh.at[pl.ds(g * CHUNK, CHUNK)], rows_v.at[0, 0])
        pltpu.sync_copy(rows_v.at[0, 0], out_hbm.at[cid, g])

  return scatter_k


def _n_groups(dout):
  p = DP // dout
  return -(-(D_EDGE + 1) // p)


def _make_edge_msg(dout, din=DP, eb=2048):
  grid = EP // eb
  ng = _n_groups(dout)
  ncol = ng * DP

  def body(ea_ref, xs_ref, w1_ref, b1_ref, w2_ref, msg_ref):
    hb = jnp.maximum(
        jnp.dot(ea_ref[...].astype(jnp.bfloat16), w1_ref[...],
                preferred_element_type=jnp.float32) + b1_ref[...], 0.0)
    y = jnp.dot(xs_ref[:, :din].astype(jnp.bfloat16), w2_ref[...],
                preferred_element_type=jnp.float32)
    acc = hb[:, :DP] * y[:, :DP]
    for g in range(1, ng):
      acc = acc + hb[:, g * DP:(g + 1) * DP] * y[:, g * DP:(g + 1) * DP]
    msg_ref[...] = acc

  def run(ea, xs, w1b, b1b, w2aug):
    return pl.pallas_call(
        body,
        grid=(grid,),
        in_specs=[
            pl.BlockSpec((eb, D_EDGE), lambda i: (i, 0)),
            pl.BlockSpec((eb, DP), lambda i: (i, 0)),
            pl.BlockSpec((D_EDGE, ncol), lambda i: (0, 0)),
            pl.BlockSpec((1, ncol), lambda i: (0, 0)),
            pl.BlockSpec((din, ncol), lambda i: (0, 0)),
        ],
        out_specs=pl.BlockSpec((eb, DP), lambda i: (i, 0)),
        out_shape=jax.ShapeDtypeStruct((EP, DP), jnp.float32),
    )(ea, xs, w1b, b1b, w2aug)

  return run


def _make_node_update(dout, nb=1000):
  grid = N // nb
  inv = 1.0 / dout

  def body(s_ref, x_ref, root_ref, bias_ref, g_ref, b_ref, out_ref):
    t = s_ref[...]
    w = DP
    while w > dout:
      w //= 2
      t = t[:, :w] + t[:, w:2 * w]
    z = (t + jnp.dot(x_ref[...], root_ref[...],
                     preferred_element_type=jnp.float32) + bias_ref[...])
    r = jnp.maximum(z, 0.0)
    mu = jnp.sum(r, axis=-1, keepdims=True) * inv
    sq = jnp.sum(r * r, axis=-1, keepdims=True) * inv
    var = sq - mu * mu
    out = (r - mu) * jax.lax.rsqrt(var + 1e-5) * g_ref[...] + b_ref[...]
    if dout < DP:
      out = jnp.concatenate(
          [out, jnp.zeros((nb, DP - dout), jnp.float32)], axis=1)
    out_ref[...] = out

  def run(s, x, root, biasr, gr, br):
    return pl.pallas_call(
        body,
        grid=(grid,),
        in_specs=[
            pl.BlockSpec((nb, DP), lambda i: (i, 0)),
            pl.BlockSpec((nb, DP), lambda i: (i, 0)),
            pl.BlockSpec((DP, dout), lambda i: (0, 0)),
            pl.BlockSpec((1, dout), lambda i: (0, 0)),
            pl.BlockSpec((1, dout), lambda i: (0, 0)),
            pl.BlockSpec((1, dout), lambda i: (0, 0)),
        ],
        out_specs=pl.BlockSpec((nb, DP), lambda i: (i, 0)),
        out_shape=jax.ShapeDtypeStruct((N, DP), jnp.float32),
    )(s, x, root, biasr, gr, br)

  return run


def _make_final(nb=1000):
  grid = N // nb

  def body(s_ref, x_ref, root_ref, bias_ref, w1_ref, b1_ref, w2_ref,
           b2_ref, emb_ref, logp_ref):
    emb = (s_ref[...]
           + jnp.dot(x_ref[...], root_ref[...],
                     preferred_element_type=jnp.float32) + bias_ref[...])
    emb_ref[...] = emb
    r = jnp.maximum(emb, 0.0)
    y = jnp.dot(r, w1_ref[...], preferred_element_type=jnp.float32) + b1_ref[...]
    y = jnp.dot(y, w2_ref[...], preferred_element_type=jnp.float32) + b2_ref[...]
    m = jnp.max(y, axis=-1, keepdims=True)
    lse = m + jnp.log(jnp.sum(jnp.exp(y - m), axis=-1, keepdims=True))
    logp_ref[...] = y - lse

  def run(s, x, root, biasr, w1, b1r, w2, b2r):
    return pl.pallas_call(
        body,
        grid=(grid,),
        in_specs=[
            pl.BlockSpec((nb, 128), lambda i: (i, 0)),
            pl.BlockSpec((nb, DP), lambda i: (i, 0)),
            pl.BlockSpec((DP, 128), lambda i: (0, 0)),
            pl.BlockSpec((1, 128), lambda i: (0, 0)),
            pl.BlockSpec((128, 64), lambda i: (0, 0)),
            pl.BlockSpec((1, 64), lambda i: (0, 0)),
            pl.BlockSpec((64, 32), lambda i: (0, 0)),
            pl.BlockSpec((1, 32), lambda i: (0, 0)),
        ],
        out_specs=[
            pl.BlockSpec((nb, 128), lambda i: (i, 0)),
            pl.BlockSpec((nb, 32), lambda i: (i, 0)),
        ],
        out_shape=[
            jax.ShapeDtypeStruct((N, 128), jnp.float32),
            jax.ShapeDtypeStruct((N, 32), jnp.float32),
        ],
    )(s, x, root, biasr, w1, b1r, w2, b2r)

  return run


def _w1aug(w1, b1, dout):
  p = DP // dout
  ng = _n_groups(dout)
  we = jnp.pad(jnp.pad(w1, ((0, 0), (0, 1))), ((0, 0), (0, ng * p - 17)))
  wb = jnp.broadcast_to(we[:, :, None], (D_EDGE, ng * p, dout))
  be = jnp.pad(jnp.concatenate([b1, jnp.ones((1,), jnp.float32)]),
               (0, ng * p - 17))
  bb = jnp.broadcast_to(be[:, None], (ng * p, dout))
  return (wb.reshape(D_EDGE, ng * DP).astype(jnp.bfloat16),
          bb.reshape(1, ng * DP))


def _w2aug(w2, b2, din, dout):
  p = DP // dout
  ng = _n_groups(dout)
  w = w2.reshape(D_EDGE, din, dout).transpose(1, 0, 2)
  w = jnp.concatenate([w, b2.reshape(din, 1, dout)], axis=1)
  w = jnp.pad(w, ((0, 0), (0, ng * p - 17), (0, 0)))
  return w.reshape(din, ng * DP).astype(jnp.bfloat16)


def _pad_cols(v, width=DP):
  return jnp.pad(v, ((0, 0), (0, width - v.shape[1])))


def _pad_vec(v, width=DP):
  return jnp.pad(v.reshape(1, -1), ((0, 0), (0, width - v.shape[0])))


def kernel(x, edge_index, edge_attr,
           en1_W1, en1_b1, en1_W2, en1_b2, root1, bias1, ln1_g, ln1_b,
           en2_W1, en2_b1, en2_W2, en2_b2, root2, bias2, ln2_g, ln2_b,
           en3_W1, en3_b1, en3_W2, en3_b2, root3, bias3,
           pm_W1, pm_b1, pm_W2, pm_b2):
  pad = EP - E
  src = jnp.concatenate(
      [edge_index[0].astype(jnp.int32),
       jnp.zeros((pad,), jnp.int32)]).reshape(NS, CPT, CHUNK)
  dst = jnp.concatenate(
      [edge_index[1].astype(jnp.int32),
       jnp.full((pad,), N, jnp.int32)])
  trash = HNP + (jnp.arange(EP, dtype=jnp.int32) % CHUNK)
  dstc = jnp.stack([
      jnp.where((dst >= c * HNP) & (dst < (c + 1) * HNP), dst - c * HNP,
                trash) for c in range(NC)
  ]).reshape(NC, NS, CPT, CHUNK)
  ea = jnp.concatenate([edge_attr, jnp.zeros((pad, D_EDGE), jnp.float32)])

  gather = _make_sc_gather()

  def r2(v):
    return v.reshape(1, -1)

  scatter = _make_sc_scatter(DP)
  zeros = jnp.zeros((CHUNK, DP), jnp.float32)

  def agg(msg):
    s = scatter(msg.reshape(NS, NBLK, CPB, CHUNK, DP), dstc, zeros)
    return s[:, :HNP // CHUNK].reshape(NP, DP)[:N]

  xp = _pad_cols(x)

  xs = gather(xp, src).reshape(EP, DP)
  msg = _make_edge_msg(32, 64, 4096)(ea, xs, *_w1aug(en1_W1, en1_b1, 32),
                               _w2aug(en1_W2, en1_b2, 64, 32))
  root1p = jnp.pad(root1, ((0, DP - 64), (0, 0)))
  h1 = _make_node_update(32)(agg(msg), xp, root1p, r2(bias1),
                             r2(ln1_g), r2(ln1_b))

  xs = gather(h1, src).reshape(EP, DP)
  msg = _make_edge_msg(64, 32, 4096)(ea, xs, *_w1aug(en2_W1, en2_b1, 64),
                               _w2aug(en2_W2, en2_b2, 32, 64))
  root2p = jnp.pad(root2, ((0, DP - 32), (0, 0)))
  h2 = _make_node_update(64)(agg(msg), h1, root2p, r2(bias2),
                             r2(ln2_g), r2(ln2_b))

  xs = gather(h2, src).reshape(EP, DP)
  msg = _make_edge_msg(128, 64)(ea, xs, *_w1aug(en3_W1, en3_b1, 128),
                                _w2aug(en3_W2, en3_b2, 64, 128))
  root3p = jnp.pad(root3, ((0, DP - 64), (0, 0)))
  emb, logp = _make_final()(agg(msg), h2, root3p, r2(bias3),
                            pm_W1, r2(pm_b1), pm_W2, r2(pm_b2))
  return (emb, logp)
